# baseline (speedup 1.0000x reference)
import functools

import jax
import jax.numpy as jnp
from jax import lax
from jax.experimental import pallas as pl
from jax.experimental.pallas import tpu as pltpu
from jax.experimental.pallas import tpu_sc as plsc
import numpy as np

EMBED = 768
PACKED = EMBED // 2
ROWS = 8 * 4608
NC, NS = 2, 16
NW = NC * NS
RPW = ROWS // NW
CHUNK = 48
NCHUNK = RPW // CHUNK
NBUF = 4

_mesh = plsc.VectorSubcoreMesh(core_axis_name="c", subcore_axis_name="s")


@functools.partial(
    pl.kernel,
    mesh=_mesh,
    out_type=jax.ShapeDtypeStruct((ROWS, PACKED), jnp.int32),
    scratch_types=[
        pltpu.VMEM((NBUF, CHUNK), jnp.int32),
        pltpu.VMEM((NBUF, CHUNK, PACKED), jnp.int32),
        pltpu.SemaphoreType.DMA((NBUF,)),
        pltpu.SemaphoreType.DMA((NBUF,)),
    ],
)
def _sc_gather(pos_hbm, w_hbm, g_hbm, idx_v, rows_v, gsem, osem):
    wid = lax.axis_index("s") * NC + lax.axis_index("c")
    base = wid * RPW

    def issue_gather(k, b):
        row0 = base + k * CHUNK
        pltpu.sync_copy(pos_hbm.at[pl.ds(row0, CHUNK)], idx_v.at[b])
        pltpu.async_copy(w_hbm.at[idx_v.at[b]], rows_v.at[b], gsem.at[b])

    for k in range(2):
        issue_gather(k, k)

    def step(k, carry):
        b = lax.rem(k, NBUF)
        row0 = base + k * CHUNK
        pltpu.make_async_copy(w_hbm.at[idx_v.at[b]], rows_v.at[b], gsem.at[b]).wait()
        pltpu.async_copy(rows_v.at[b], g_hbm.at[pl.ds(row0, CHUNK)], osem.at[b])

        b2 = lax.rem(k + 2, NBUF)

        @pl.when(k >= 2)
        def _():
            row0p = base + (k - 2) * CHUNK
            pltpu.make_async_copy(
                rows_v.at[b2], g_hbm.at[pl.ds(row0p, CHUNK)], osem.at[b2]
            ).wait()

        @pl.when(k + 2 < NCHUNK)
        def _():
            issue_gather(k + 2, b2)

        return carry

    lax.fori_loop(0, NCHUNK, step, 0)

    for k in range(NCHUNK - 2, NCHUNK):
        b = k % NBUF
        row0 = base + k * CHUNK
        pltpu.make_async_copy(
            rows_v.at[b], g_hbm.at[pl.ds(row0, CHUNK)], osem.at[b]
        ).wait()


BL = 1024


def _tc_add_body(x_ref, g_ref, o_ref):
    g = g_ref[...]
    lo = lax.bitcast_convert_type(g << 16, jnp.float32)
    hi = lax.bitcast_convert_type(g & jnp.int32(-65536), jnp.float32)
    o_ref[:, :PACKED] = x_ref[:, :PACKED] + lo
    o_ref[:, PACKED:] = x_ref[:, PACKED:] + hi


def _tc_add(xf, g):
    n = xf.shape[0]
    return pl.pallas_call(
        _tc_add_body,
        grid=(n // BL,),
        in_specs=[
            pl.BlockSpec((BL, EMBED), lambda i: (i, 0)),
            pl.BlockSpec((BL, PACKED), lambda i: (i, 0)),
        ],
        out_specs=pl.BlockSpec((BL, EMBED), lambda i: (i, 0)),
        out_shape=jax.ShapeDtypeStruct((n, EMBED), jnp.float32),
    )(xf, g)


def kernel(x, pos_ids, weight):
    b, l, d = x.shape
    assert b * l == ROWS and d == EMBED
    w_packed = lax.bitcast_convert_type(
        weight.astype(jnp.bfloat16).reshape(-1, 2, PACKED).transpose(0, 2, 1),
        jnp.int32,
    )
    g = _sc_gather(pos_ids.reshape(ROWS), w_packed)
    out = _tc_add(x.reshape(ROWS, EMBED), g)
    return out.reshape(x.shape)
